# baseline (speedup 1.0000x reference)
import functools

import jax
import jax.numpy as jnp
from jax import lax
from jax.experimental import pallas as pl
from jax.experimental.pallas import tpu as pltpu
from jax.experimental.pallas import tpu_sc as plsc

_F32 = jnp.float32


def _sc_gather(table, idx):
    n = idx.shape[0]
    h_dim = table.shape[1]
    info = plsc.get_sparse_core_info()
    nw = info.num_cores * info.num_subcores
    ch = 128
    n_chunks = n // ch
    iters = (n_chunks + nw - 1) // nw

    mesh = plsc.VectorSubcoreMesh(core_axis_name="c", subcore_axis_name="s")

    @functools.partial(
        pl.kernel,
        mesh=mesh,
        out_type=jax.ShapeDtypeStruct((n, h_dim), _F32),
        scratch_types=[
            pltpu.VMEM((ch,), jnp.int32),
            pltpu.VMEM((ch, h_dim), _F32),
            pltpu.SemaphoreType.DMA,
        ],
        compiler_params=pltpu.CompilerParams(use_tc_tiling_on_sc=False),
    )
    def k(table_hbm, idx_hbm, out_hbm, idx_v, rows_v, sem):
        wid = lax.axis_index("s") * info.num_cores + lax.axis_index("c")

        def step(i, carry):
            c = wid + i * nw

            @pl.when(c < n_chunks)
            def _():
                base = c * ch
                pltpu.sync_copy(idx_hbm.at[pl.ds(base, ch)], idx_v)
                pltpu.async_copy(table_hbm.at[idx_v], rows_v, sem).wait()
                pltpu.sync_copy(rows_v, out_hbm.at[pl.ds(base, ch)])

            return carry

        lax.fori_loop(0, iters, step, 0)

    return k(table, idx)


def _dot(a, b, dims):
    return lax.dot_general(a, b, (dims, ((), ())),
                           preferred_element_type=_F32,
                           precision=lax.Precision.HIGHEST)


def _gat_block(x, W, a1, a2, evalid, concat):
    Wh = _dot(x, W, ((1,), (0,)))
    e1 = _dot(Wh, a1, ((1,), (1,)))
    e2 = _dot(a2, Wh, ((1,), (1,)))
    e = e1 + e2
    e = jnp.where(e >= 0, e, 0.2 * e)
    e = jnp.where(evalid, e, -1e9)
    e = e - jnp.max(e, axis=1, keepdims=True)
    p = jnp.exp(e)
    p = jnp.where(evalid, p, 0.0)
    s = jnp.sum(p, axis=1, keepdims=True)
    out = _dot(p / s, Wh, ((1,), (0,)))
    if concat:
        out = jnp.where(out > 0, out, jnp.exp(jnp.minimum(out, 0.0)) - 1.0)
    return out


def _gat_stack(h, mask, W_in, a_in, W_out, a_out, w, W1, b1, W2, b2, W3, Wt, bt,
               bb=8):
    B, L = mask.shape
    H = h.shape[-1]
    NH = W_in.shape[0]
    M = bb * L
    nblk = B // bb

    h3 = h.reshape(nblk, M, H)
    mask_a = mask.reshape(nblk, bb, L)
    mask_b = mask.reshape(nblk, M, 1)
    a_in_r = a_in[:, :, 0]
    a_out_r = a_out.reshape(1, 2 * H)

    def body(h_ref, ma_ref, mb_ref, wi_ref, ai_ref, wo_ref, ao_ref, w_ref,
             w1_ref, b1_ref, w2_ref, b2_ref, w3_ref, wt_ref, bt_ref, out_ref):
        hx = h_ref[0]
        maskb = ma_ref[0]
        mask_col = mb_ref[0]

        iota_b = lax.broadcasted_iota(jnp.int32, (bb, M), 0).astype(_F32)
        iota_j = lax.broadcasted_iota(jnp.int32, (bb, M), 1).astype(_F32)
        lo = iota_b * L
        bmat = ((iota_j >= lo) & (iota_j < lo + L)).astype(_F32)

        seqb = jnp.sum(maskb, axis=1, keepdims=True)
        seq_col = _dot(seqb, bmat, ((0,), (0,)))
        brow = lax.broadcasted_iota(jnp.int32, (1, bb), 1).astype(_F32)
        batch_row = _dot(brow, bmat, ((1,), (0,)))
        jrow = lax.broadcasted_iota(jnp.int32, (1, M), 1).astype(_F32)
        pos_row = jrow - L * batch_row
        validf = (pos_row < seq_col).astype(_F32)
        bv = bmat * validf
        evalid = _dot(bmat, bv, ((0,), (0,))) > 0.5

        heads = []
        for i in range(NH):
            a1 = ai_ref[i:i + 1, 0:H]
            a2 = ai_ref[i:i + 1, H:2 * H]
            wi = wi_ref[i]
            heads.append(_gat_block(hx, wi, a1, a2, evalid, True))
        mul_seq = jnp.concatenate(heads, axis=1)
        z = _dot(mul_seq, w_ref[...], ((1,), (0,)))
        mul_one = jnp.where(z > 0, z, jnp.exp(jnp.minimum(z, 0.0)) - 1.0)

        ao1 = ao_ref[0:1, 0:H]
        ao2 = ao_ref[0:1, H:2 * H]
        seq_hidden = _gat_block(mul_one, wo_ref[...], ao1, ao2, evalid, False)

        sel = (iota_j == (lo + seqb - 1.0)).astype(_F32)
        ht = _dot(sel, seq_hidden, ((1,), (0,)))
        q1 = _dot(ht, w1_ref[...], ((1,), (0,))) + b1_ref[...]
        q1f = _dot(bmat, q1, ((0,), (0,)))
        q2 = _dot(seq_hidden, w2_ref[...], ((1,), (0,))) + b2_ref[...]
        zs = q1f + q2
        sig = 1.0 / (1.0 + jnp.exp(-zs))
        alpha = _dot(sig, w3_ref[...], ((1,), (0,)))
        wgt = alpha * seq_hidden * mask_col
        apool = _dot(bmat, wgt, ((1,), (0,)))
        cat = jnp.concatenate([apool, ht], axis=1)
        out_ref[0] = _dot(cat, wt_ref[...], ((1,), (0,))) + bt_ref[...]

    def const(*s):
        return pl.BlockSpec(s, lambda i: tuple(0 for _ in s))

    a3 = pl.pallas_call(
        body,
        grid=(nblk,),
        in_specs=[
            pl.BlockSpec((1, M, H), lambda i: (i, 0, 0)),
            pl.BlockSpec((1, bb, L), lambda i: (i, 0, 0)),
            pl.BlockSpec((1, M, 1), lambda i: (i, 0, 0)),
            const(NH, H, H),
            const(NH, 2 * H),
            const(H, H),
            const(1, 2 * H),
            const(NH * H, H),
            const(H, H),
            const(1, H),
            const(H, H),
            const(1, H),
            const(H, 1),
            const(2 * H, H),
            const(1, H),
        ],
        out_specs=pl.BlockSpec((1, bb, H), lambda i: (i, 0, 0)),
        out_shape=jax.ShapeDtypeStruct((nblk, bb, H), _F32),
        compiler_params=pltpu.CompilerParams(
            dimension_semantics=("arbitrary",)),
    )(h3, mask_a, mask_b, W_in, a_in_r, W_out, a_out_r, w,
      W1, b1.reshape(1, H), W2, b2.reshape(1, H), W3, Wt, bt.reshape(1, H))
    return a3.reshape(B, H)


def _scores_mm(a_vec, table, flagf, V, vt=1024):
    B, H = a_vec.shape
    ngrid = (V + vt - 1) // vt

    def body(a_ref, t_ref, f_ref, o_ref):
        o_ref[...] = lax.dot_general(
            a_ref[...], t_ref[...], (((1,), (1,)), ((), ())),
            preferred_element_type=_F32) + f_ref[...]

    return pl.pallas_call(
        body,
        grid=(ngrid,),
        in_specs=[
            pl.BlockSpec((B, H), lambda j: (0, 0)),
            pl.BlockSpec((vt, H), lambda j: (j, 0)),
            pl.BlockSpec((1, 1), lambda j: (0, 0)),
        ],
        out_specs=pl.BlockSpec((B, vt), lambda j: (0, j)),
        out_shape=jax.ShapeDtypeStruct((B, V), _F32),
        compiler_params=pltpu.CompilerParams(
            dimension_semantics=("arbitrary",)),
    )(a_vec, table, flagf)


def kernel(path, mask, targets, flag, user_embedding, W_in, a_in, W_out, a_out,
           w, W1, b1, W2, b2, W3, Wt, bt):
    B, L = path.shape
    V = user_embedding.shape[0] - 1
    idx = path.reshape(-1).astype(jnp.int32)
    h = _sc_gather(user_embedding, idx)
    a_vec = _gat_stack(h, mask, W_in, a_in, W_out, a_out, w,
                       W1, b1, W2, b2, W3, Wt, bt)
    flagf = jnp.asarray(flag, _F32).reshape(1, 1)
    scores = _scores_mm(a_vec, user_embedding, flagf, V)
    return (scores, targets)
